# Initial kernel scaffold; baseline (speedup 1.0000x reference)
import functools

import jax
import jax.numpy as jnp
from jax import lax
from jax.experimental import pallas as pl
from jax.experimental.pallas import tpu as pltpu
from jax.experimental.pallas import tpu_sc as plsc

N = 10000
E = 320000
D = 128
G = 64
EPS = 1e-5

NC = 2
NS = 16
NW = NC * NS
EB = 128
NBLK = E // EB
NP = 10240
RPT = NP // NS
BR = 1024
_MESH = plsc.VectorSubcoreMesh(core_axis_name="c", subcore_axis_name="s")


def _worker_id():
    return lax.axis_index("s") * NC + lax.axis_index("c")


def _num_blocks(wid):
    return NBLK // NW + jnp.where(wid < (NBLK % NW), 1, 0)


@functools.partial(
    pl.kernel,
    out_type=jax.ShapeDtypeStruct((NC, NP), jnp.float32),
    mesh=_MESH,
    scratch_types=[
        pltpu.VMEM((EB,), jnp.int32),
        pltpu.VMEM((EB,), jnp.float32),
        pltpu.VMEM_SHARED((NP,), jnp.float32),
    ],
)
def _sc_degree(dst_hbm, ew_hbm, zeros_hbm, out_hbm, dst_v, ew_v, acc):
    c = lax.axis_index("c")
    s = lax.axis_index("s")
    wid = _worker_id()
    r0 = s * RPT
    pltpu.sync_copy(zeros_hbm.at[pl.ds(r0, RPT)], acc.at[pl.ds(r0, RPT)])
    plsc.subcore_barrier()

    def body(i, _):
        off = (wid + i * NW) * EB
        pltpu.sync_copy(dst_hbm.at[pl.ds(off, EB)], dst_v)
        pltpu.sync_copy(ew_hbm.at[pl.ds(off, EB)], ew_v)
        pltpu.sync_copy(ew_v, acc.at[dst_v], add=True)
        return 0

    lax.fori_loop(0, _num_blocks(wid), body, 0)
    plsc.subcore_barrier()
    pltpu.sync_copy(acc.at[pl.ds(r0, RPT)], out_hbm.at[c, pl.ds(r0, RPT)])


@functools.partial(
    pl.kernel,
    out_type=jax.ShapeDtypeStruct((NC, NP, D), jnp.float32),
    mesh=_MESH,
    scratch_types=[
        pltpu.VMEM((EB,), jnp.int32),
        pltpu.VMEM((EB,), jnp.int32),
        pltpu.VMEM((EB,), jnp.float32),
        pltpu.VMEM((EB, D), jnp.float32),
        pltpu.VMEM_SHARED((NP, D), jnp.float32),
        pltpu.SemaphoreType.DMA,
    ],
)
def _sc_aggregate(hs_hbm, src_hbm, dst_hbm, ew_hbm, zeros_hbm, out_hbm,
                  src_v, dst_v, ew_v, rows, acc, sem):
    c = lax.axis_index("c")
    s = lax.axis_index("s")
    wid = _worker_id()
    r0 = s * RPT
    pltpu.sync_copy(zeros_hbm.at[pl.ds(r0, RPT)], acc.at[pl.ds(r0, RPT)])
    plsc.subcore_barrier()

    def body(i, _):
        off = (wid + i * NW) * EB
        pltpu.sync_copy(src_hbm.at[pl.ds(off, EB)], src_v)
        pltpu.sync_copy(dst_hbm.at[pl.ds(off, EB)], dst_v)
        pltpu.sync_copy(ew_hbm.at[pl.ds(off, EB)], ew_v)
        pltpu.async_copy(hs_hbm.at[src_v], rows, sem).wait()

        def scale(k, _):
            w = ew_v[k]
            for j in range(D // 16):
                sl = pl.ds(j * 16, 16)
                rows[k, sl] = rows[k, sl] * w
            return 0

        lax.fori_loop(0, EB, scale, 0)
        pltpu.sync_copy(rows, acc.at[dst_v], add=True)
        return 0

    lax.fori_loop(0, _num_blocks(wid), body, 0)
    plsc.subcore_barrier()
    pltpu.sync_copy(acc.at[pl.ds(r0, RPT)], out_hbm.at[c, pl.ds(r0, RPT)])


def _tc1_body(x_ref, w_ref, d0_ref, d1_ref, hs_ref, dis_ref):
    deg = 1.0 + d0_ref[...] + d1_ref[...]
    dis = lax.rsqrt(jnp.maximum(deg, 1e-12))
    h = jnp.dot(x_ref[...], w_ref[...], preferred_element_type=jnp.float32)
    hs_ref[...] = h * dis
    dis_ref[...] = dis


def _tc2_body(a0_ref, a1_ref, hs_ref, dis_ref, w_ref, s1_ref, c1_ref, out_ref):
    dis = dis_ref[...]
    conv = (a0_ref[...] + a1_ref[...] + hs_ref[...]) * dis
    h2 = jnp.maximum(conv * s1_ref[...] + c1_ref[...], 0.0)
    out_ref[...] = jnp.dot(h2, w_ref[...],
                           preferred_element_type=jnp.float32) * dis


def _tc3_body(a0_ref, a1_ref, hs_ref, dis_ref, s2_ref, c2_ref, b_ref,
              wout_ref, bout_ref, out_ref, sums_ref, cnt_ref):
    i = pl.program_id(0)

    @pl.when(i == 0)
    def _():
        sums_ref[...] = jnp.zeros_like(sums_ref)
        cnt_ref[...] = jnp.zeros_like(cnt_ref)

    dis = dis_ref[...]
    conv = (a0_ref[...] + a1_ref[...] + hs_ref[...]) * dis
    hfin = jnp.maximum(conv * s2_ref[...] + c2_ref[...], 0.0)
    gids = lax.broadcasted_iota(jnp.float32, (BR, G), 1)
    onehot = (b_ref[...] == gids).astype(jnp.float32)
    dn = (((0,), (0,)), ((), ()))
    sums_ref[...] += lax.dot_general(onehot, hfin, dn,
                                     preferred_element_type=jnp.float32)
    cnt_ref[...] += lax.dot_general(onehot, jnp.ones((BR, 1), jnp.float32),
                                    dn, preferred_element_type=jnp.float32)

    @pl.when(i == NP // BR - 1)
    def _():
        pooled = sums_ref[...] / jnp.maximum(cnt_ref[...], 1.0)
        out_ref[...] = jnp.dot(pooled, wout_ref[...],
                               preferred_element_type=jnp.float32) + bout_ref[...]


def _row_spec(dim):
    return pl.BlockSpec((BR, dim), lambda i: (i, 0))


def _full_spec(r, c):
    return pl.BlockSpec((r, c), lambda i: (0, 0))


def kernel(x, edge_index, edge_weight, batch, W1, b1, g1, be1,
           W2, b2, g2, be2, Wout, bout):
    f32 = jnp.float32
    src = edge_index[0]
    dst = edge_index[1]

    xp = jnp.zeros((NP, D), f32).at[:N].set(x)
    batchf = jnp.full((NP, 1), float(G), f32).at[:N, 0].set(batch.astype(f32))
    zeros1 = jnp.zeros((NP,), f32)
    zeros2 = jnp.zeros((NP, D), f32)

    sc = 1.0 / jnp.sqrt(jnp.float32(1.0 + EPS))
    s1 = (g1 * sc).reshape(1, D)
    c1 = (b1 * g1 * sc + be1).reshape(1, D)
    s2 = (g2 * sc).reshape(1, D)
    c2 = (b2 * g2 * sc + be2).reshape(1, D)

    degp = _sc_degree(dst, edge_weight, zeros1)
    d0 = degp[0].reshape(NP, 1)
    d1 = degp[1].reshape(NP, 1)

    grid = (NP // BR,)
    hs1, dis = pl.pallas_call(
        _tc1_body,
        grid=grid,
        in_specs=[_row_spec(D), _full_spec(D, D), _row_spec(1), _row_spec(1)],
        out_specs=[_row_spec(D), _row_spec(1)],
        out_shape=[jax.ShapeDtypeStruct((NP, D), f32),
                   jax.ShapeDtypeStruct((NP, 1), f32)],
    )(xp, W1, d0, d1)

    accp1 = _sc_aggregate(hs1, src, dst, edge_weight, zeros2)

    hs2 = pl.pallas_call(
        _tc2_body,
        grid=grid,
        in_specs=[_row_spec(D), _row_spec(D), _row_spec(D), _row_spec(1),
                  _full_spec(D, D), _full_spec(1, D), _full_spec(1, D)],
        out_specs=_row_spec(D),
        out_shape=jax.ShapeDtypeStruct((NP, D), f32),
    )(accp1[0], accp1[1], hs1, dis, W2, s1, c1)

    accp2 = _sc_aggregate(hs2, src, dst, edge_weight, zeros2)

    out = pl.pallas_call(
        _tc3_body,
        grid=grid,
        in_specs=[_row_spec(D), _row_spec(D), _row_spec(D), _row_spec(1),
                  _full_spec(1, D), _full_spec(1, D), _row_spec(1),
                  _full_spec(D, 1), _full_spec(1, 1)],
        out_specs=pl.BlockSpec((G, 1), lambda i: (0, 0)),
        out_shape=jax.ShapeDtypeStruct((G, 1), f32),
        scratch_shapes=[pltpu.VMEM((G, D), f32), pltpu.VMEM((G, 1), f32)],
    )(accp2[0], accp2[1], hs2, dis, s2, c2, batchf,
      Wout, bout.reshape(1, 1))
    return out


# speedup vs baseline: 11.8705x; 11.8705x over previous
import functools

import jax
import jax.numpy as jnp
from jax import lax
from jax.experimental import pallas as pl
from jax.experimental.pallas import tpu as pltpu
from jax.experimental.pallas import tpu_sc as plsc

N = 10000
E = 320000
D = 128
G = 64
EPS = 1e-5

NC = 2
NS = 16
NW = NC * NS
EB = 128
NBLK = E // EB
NP = 10240
RPT = NP // NS
BR = 1024
def _worker_id():
    return lax.axis_index("s") * NC + lax.axis_index("c")


def _num_blocks(wid):
    return NBLK // NW + jnp.where(wid < (NBLK % NW), 1, 0)


def _sc_degree_body(dst_hbm, ew_hbm, zeros_hbm, out_hbm, dst_v, ew_v, acc):
    c = lax.axis_index("c")
    s = lax.axis_index("s")
    wid = _worker_id()
    r0 = s * RPT
    pltpu.sync_copy(zeros_hbm.at[pl.ds(r0, RPT)], acc.at[pl.ds(r0, RPT)])
    plsc.subcore_barrier()

    def body(i, _):
        off = (wid + i * NW) * EB
        pltpu.sync_copy(dst_hbm.at[pl.ds(off, EB)], dst_v)
        pltpu.sync_copy(ew_hbm.at[pl.ds(off, EB)], ew_v)
        pltpu.sync_copy(ew_v, acc.at[dst_v], add=True)
        return 0

    lax.fori_loop(0, _num_blocks(wid), body, 0)
    plsc.subcore_barrier()
    pltpu.sync_copy(acc.at[pl.ds(r0, RPT)], out_hbm.at[c, pl.ds(r0, RPT)])


def _sc_aggregate_body(hs_hbm, src_hbm, dst_hbm, ew_hbm, zeros_hbm, out_hbm,
                       src_v, dst_v, ew_v, rows, acc, sem):
    c = lax.axis_index("c")
    s = lax.axis_index("s")
    wid = _worker_id()
    r0 = s * RPT
    pltpu.sync_copy(zeros_hbm.at[pl.ds(r0, RPT)], acc.at[pl.ds(r0, RPT)])
    plsc.subcore_barrier()

    def body(i, _):
        off = (wid + i * NW) * EB
        pltpu.sync_copy(src_hbm.at[pl.ds(off, EB)], src_v)
        pltpu.sync_copy(dst_hbm.at[pl.ds(off, EB)], dst_v)
        pltpu.sync_copy(ew_hbm.at[pl.ds(off, EB)], ew_v)
        pltpu.async_copy(hs_hbm.at[src_v], rows, sem).wait()

        def scale(g, _):
            wv = ew_v[pl.ds(g * 16, 16)]
            for k in range(16):
                w = wv[k]
                r = g * 16 + k
                for j in range(D // 16):
                    sl = pl.ds(j * 16, 16)
                    rows[r, sl] = rows[r, sl] * w
            return 0

        lax.fori_loop(0, EB // 16, scale, 0)
        pltpu.sync_copy(rows, acc.at[dst_v], add=True)
        return 0

    lax.fori_loop(0, _num_blocks(wid), body, 0)
    plsc.subcore_barrier()
    pltpu.sync_copy(acc.at[pl.ds(r0, RPT)], out_hbm.at[c, pl.ds(r0, RPT)])


@functools.cache
def _sc_kernels():
    mesh = plsc.VectorSubcoreMesh(core_axis_name="c", subcore_axis_name="s",
                                  num_cores=NC, num_subcores=NS)
    deg = pl.kernel(
        _sc_degree_body,
        out_type=jax.ShapeDtypeStruct((NC, NP), jnp.float32),
        mesh=mesh,
        scratch_types=[
            pltpu.VMEM((EB,), jnp.int32),
            pltpu.VMEM((EB,), jnp.float32),
            pltpu.VMEM_SHARED((NP,), jnp.float32),
        ],
    )
    agg = pl.kernel(
        _sc_aggregate_body,
        out_type=jax.ShapeDtypeStruct((NC, NP, D), jnp.float32),
        mesh=mesh,
        scratch_types=[
            pltpu.VMEM((EB,), jnp.int32),
            pltpu.VMEM((EB,), jnp.int32),
            pltpu.VMEM((EB,), jnp.float32),
            pltpu.VMEM((EB, D), jnp.float32),
            pltpu.VMEM_SHARED((NP, D), jnp.float32),
            pltpu.SemaphoreType.DMA,
        ],
    )
    return deg, agg


def _sc_degree(dst, ew, zeros1):
    return _sc_kernels()[0](dst, ew, zeros1)


def _sc_aggregate(hs, src, dst, ew, zeros2):
    return _sc_kernels()[1](hs, src, dst, ew, zeros2)


def _tc1_body(x_ref, w_ref, d0_ref, d1_ref, hs_ref, dis_ref):
    deg = 1.0 + d0_ref[...] + d1_ref[...]
    dis = lax.rsqrt(jnp.maximum(deg, 1e-12))
    h = jnp.dot(x_ref[...], w_ref[...], preferred_element_type=jnp.float32)
    hs_ref[...] = h * dis
    dis_ref[...] = dis


def _tc2_body(a0_ref, a1_ref, hs_ref, dis_ref, w_ref, s1_ref, c1_ref, out_ref):
    dis = dis_ref[...]
    conv = (a0_ref[...] + a1_ref[...] + hs_ref[...]) * dis
    h2 = jnp.maximum(conv * s1_ref[...] + c1_ref[...], 0.0)
    out_ref[...] = jnp.dot(h2, w_ref[...],
                           preferred_element_type=jnp.float32) * dis


def _tc3_body(a0_ref, a1_ref, hs_ref, dis_ref, s2_ref, c2_ref, b_ref,
              wout_ref, bout_ref, out_ref, sums_ref, cnt_ref):
    i = pl.program_id(0)

    @pl.when(i == 0)
    def _():
        sums_ref[...] = jnp.zeros_like(sums_ref)
        cnt_ref[...] = jnp.zeros_like(cnt_ref)

    dis = dis_ref[...]
    conv = (a0_ref[...] + a1_ref[...] + hs_ref[...]) * dis
    hfin = jnp.maximum(conv * s2_ref[...] + c2_ref[...], 0.0)
    gids = lax.broadcasted_iota(jnp.int32, (BR, G), 1).astype(jnp.float32)
    onehot = (b_ref[...] == gids).astype(jnp.float32)
    dn = (((0,), (0,)), ((), ()))
    sums_ref[...] += lax.dot_general(onehot, hfin, dn,
                                     preferred_element_type=jnp.float32)
    cnt_ref[...] += lax.dot_general(onehot, jnp.ones((BR, 1), jnp.float32),
                                    dn, preferred_element_type=jnp.float32)

    @pl.when(i == NP // BR - 1)
    def _():
        pooled = sums_ref[...] / jnp.maximum(cnt_ref[...], 1.0)
        out_ref[...] = jnp.dot(pooled, wout_ref[...],
                               preferred_element_type=jnp.float32) + bout_ref[...]


def _row_spec(dim):
    return pl.BlockSpec((BR, dim), lambda i: (i, 0))


def _full_spec(r, c):
    return pl.BlockSpec((r, c), lambda i: (0, 0))


def kernel(x, edge_index, edge_weight, batch, W1, b1, g1, be1,
           W2, b2, g2, be2, Wout, bout):
    f32 = jnp.float32
    src = edge_index[0]
    dst = edge_index[1]

    xp = jnp.zeros((NP, D), f32).at[:N].set(x)
    batchf = jnp.full((NP, 1), float(G), f32).at[:N, 0].set(batch.astype(f32))
    zeros1 = jnp.zeros((NP,), f32)
    zeros2 = jnp.zeros((NP, D), f32)

    sc = 1.0 / jnp.sqrt(jnp.float32(1.0 + EPS))
    s1 = (g1 * sc).reshape(1, D)
    c1 = (b1 * g1 * sc + be1).reshape(1, D)
    s2 = (g2 * sc).reshape(1, D)
    c2 = (b2 * g2 * sc + be2).reshape(1, D)

    degp = _sc_degree(dst, edge_weight, zeros1)
    d0 = degp[0].reshape(NP, 1)
    d1 = degp[1].reshape(NP, 1)

    grid = (NP // BR,)
    hs1, dis = pl.pallas_call(
        _tc1_body,
        grid=grid,
        in_specs=[_row_spec(D), _full_spec(D, D), _row_spec(1), _row_spec(1)],
        out_specs=[_row_spec(D), _row_spec(1)],
        out_shape=[jax.ShapeDtypeStruct((NP, D), f32),
                   jax.ShapeDtypeStruct((NP, 1), f32)],
    )(xp, W1, d0, d1)

    accp1 = _sc_aggregate(hs1, src, dst, edge_weight, zeros2)

    hs2 = pl.pallas_call(
        _tc2_body,
        grid=grid,
        in_specs=[_row_spec(D), _row_spec(D), _row_spec(D), _row_spec(1),
                  _full_spec(D, D), _full_spec(1, D), _full_spec(1, D)],
        out_specs=_row_spec(D),
        out_shape=jax.ShapeDtypeStruct((NP, D), f32),
    )(accp1[0], accp1[1], hs1, dis, W2, s1, c1)

    accp2 = _sc_aggregate(hs2, src, dst, edge_weight, zeros2)

    out = pl.pallas_call(
        _tc3_body,
        grid=grid,
        in_specs=[_row_spec(D), _row_spec(D), _row_spec(D), _row_spec(1),
                  _full_spec(1, D), _full_spec(1, D), _row_spec(1),
                  _full_spec(D, 1), _full_spec(1, 1)],
        out_specs=pl.BlockSpec((G, 1), lambda i: (0, 0)),
        out_shape=jax.ShapeDtypeStruct((G, 1), f32),
        scratch_shapes=[pltpu.VMEM((G, D), f32), pltpu.VMEM((G, 1), f32)],
    )(accp2[0], accp2[1], hs2, dis, s2, c2, batchf,
      Wout, bout.reshape(1, 1))
    return out
